# baseline (speedup 1.0000x reference)
import functools

import jax
import jax.numpy as jnp
from jax.experimental import pallas as pl

_B = 16
_N = 1024
_NSAMPLE = 32
_F32 = jnp.float32


def _relu(x):
    return jnp.maximum(x, 0.0)


def _cumsum_lanes(x, width):
    shift = 1
    q = x.shape[0]
    while shift < width:
        z = jnp.zeros((q, shift), dtype=x.dtype)
        x = x + jnp.concatenate([z, x[:, : width - shift]], axis=1)
        shift *= 2
    return x


def _fps_body(xyzt_ref, out_ref, *, n, npoint):
    x0 = xyzt_ref[0]
    x1 = xyzt_ref[1]
    x2 = xyzt_ref[2]
    iota_n = jax.lax.broadcasted_iota(jnp.int32, (_B, n), 1).astype(_F32)
    iota_np = jax.lax.broadcasted_iota(jnp.int32, (_B, npoint), 1).astype(_F32)

    def body(i, carry):
        dists, far, nx0, nx1, nx2 = carry
        onehot = iota_n == far
        c0 = jnp.sum(jnp.where(onehot, x0, 0.0), axis=1, keepdims=True)
        c1 = jnp.sum(jnp.where(onehot, x1, 0.0), axis=1, keepdims=True)
        c2 = jnp.sum(jnp.where(onehot, x2, 0.0), axis=1, keepdims=True)
        fi = i.astype(_F32)
        sel = iota_np == fi
        nx0 = jnp.where(sel, c0, nx0)
        nx1 = jnp.where(sel, c1, nx1)
        nx2 = jnp.where(sel, c2, nx2)
        d0 = x0 - c0
        d1 = x1 - c1
        d2 = x2 - c2
        d = d0 * d0 + d1 * d1 + d2 * d2
        dists = jnp.minimum(dists, d)
        m = jnp.max(dists, axis=1, keepdims=True)
        cand = jnp.where(dists == m, iota_n, float(n))
        far = jnp.min(cand, axis=1, keepdims=True)
        return (dists, far, nx0, nx1, nx2)

    init = (
        jnp.full((_B, n), 1e10, dtype=_F32),
        jnp.zeros((_B, 1), dtype=_F32),
        jnp.zeros((_B, npoint), dtype=_F32),
        jnp.zeros((_B, npoint), dtype=_F32),
        jnp.zeros((_B, npoint), dtype=_F32),
    )
    _, _, nx0, nx1, nx2 = jax.lax.fori_loop(0, npoint, body, init)
    out_ref[0] = nx0
    out_ref[1] = nx1
    out_ref[2] = nx2


def _fps(xyz, npoint):
    n = xyz.shape[1]
    xyzt = jnp.transpose(xyz, (2, 0, 1))
    out = pl.pallas_call(
        functools.partial(_fps_body, n=n, npoint=npoint),
        out_shape=jax.ShapeDtypeStruct((3, _B, npoint), _F32),
    )(xyzt)
    return jnp.transpose(out, (1, 2, 0))


def _sa_body(xyzt_ref, g_ref, new_ref, npad_ref, w1_ref, b1_ref, w2_ref,
             b2_ref, w3_ref, b3_ref, out_ref, *, r2, n):
    xt = xyzt_ref[0]
    gsrc = g_ref[0]
    new = new_ref[0]
    npad = npad_ref[0]
    w1 = w1_ref[:]
    b1 = b1_ref[:]
    w2 = w2_ref[:]
    b2 = b2_ref[:]
    w3 = w3_ref[:]
    b3 = b3_ref[:]

    q = new.shape[0]
    ch = gsrc.shape[1]
    t0 = xt[0:1, :]
    t1 = xt[1:2, :]
    t2 = xt[2:3, :]
    nsrc = t0 * t0 + t1 * t1 + t2 * t2
    nnew = jnp.sum(new * new, axis=1, keepdims=True)
    d = jnp.dot(new, xt, preferred_element_type=_F32)
    sqd = (nnew + nsrc) - 2.0 * d
    mask = sqd <= r2
    mf = jnp.where(mask, 1.0, 0.0).astype(_F32)
    tri = jnp.where(
        jax.lax.broadcasted_iota(jnp.int32, (n, n), 0)
        <= jax.lax.broadcasted_iota(jnp.int32, (n, n), 1),
        1.0, 0.0).astype(_F32)
    cnt = jnp.dot(mf, tri, preferred_element_type=_F32)
    count = cnt[:, n - 1 : n]

    def mlp3(h):
        a = _relu(jnp.dot(h, w1, preferred_element_type=_F32) + b1)
        a = _relu(jnp.dot(a, w2, preferred_element_type=_F32) + b2)
        a = _relu(jnp.dot(a, w3, preferred_element_type=_F32) + b3)
        return a

    cnt3 = cnt.reshape(1, q, n)
    mask3 = mask.reshape(1, q, n)
    count3 = count.reshape(1, q, 1)
    npad3 = npad.reshape(1, q, ch)
    sb = 8
    acc = None
    g0 = None
    for blk in range(_NSAMPLE // sb):
        kf3 = (
            jax.lax.broadcasted_iota(jnp.int32, (sb, 1, 1), 0).astype(_F32)
            + (1.0 + sb * blk)
        )
        sel3 = jnp.where(
            jnp.logical_and(mask3, cnt3 == kf3), 1.0, 0.0).astype(_F32)
        g = jnp.dot(sel3.reshape(sb * q, n), gsrc,
                    preferred_element_type=_F32,
                    precision=jax.lax.Precision.HIGHEST)
        g3 = g.reshape(sb, q, ch)
        if blk == 0:
            g0 = jnp.where(count >= 1.0, g3[0], gsrc[0:1, :])
        g3 = jnp.where(count3 >= kf3, g3, g0.reshape(1, q, ch))
        a = mlp3((g3 - npad3).reshape(sb * q, ch))
        amax = jnp.max(a.reshape(sb, q, a.shape[1]), axis=0)
        acc = amax if acc is None else jnp.maximum(acc, amax)
    out_ref[0] = acc


def _sa(src_xyz, new_xyz, feats, layers, radius, qchunk):
    n = src_xyz.shape[1]
    s = new_xyz.shape[1]
    c = feats.shape[2]
    ch = 3 + c
    (w1, b1), (w2, b2), (w3, b3) = layers
    cout = w3.shape[1]
    gsrc = jnp.concatenate([src_xyz, feats], axis=-1)
    xyzt = jnp.transpose(src_xyz, (0, 2, 1))
    npad = jnp.concatenate(
        [new_xyz, jnp.zeros((_B, s, c), dtype=_F32)], axis=-1
    )
    nq = s // qchunk
    grid = (_B, nq)
    out = pl.pallas_call(
        functools.partial(_sa_body, r2=radius * radius, n=n),
        grid=grid,
        in_specs=[
            pl.BlockSpec((1, 3, n), lambda b, q: (b, 0, 0)),
            pl.BlockSpec((1, n, ch), lambda b, q: (b, 0, 0)),
            pl.BlockSpec((1, qchunk, 3), lambda b, q: (b, q, 0)),
            pl.BlockSpec((1, qchunk, ch), lambda b, q: (b, q, 0)),
            pl.BlockSpec(w1.shape, lambda b, q: (0, 0)),
            pl.BlockSpec((1, b1.shape[0]), lambda b, q: (0, 0)),
            pl.BlockSpec(w2.shape, lambda b, q: (0, 0)),
            pl.BlockSpec((1, b2.shape[0]), lambda b, q: (0, 0)),
            pl.BlockSpec(w3.shape, lambda b, q: (0, 0)),
            pl.BlockSpec((1, b3.shape[0]), lambda b, q: (0, 0)),
        ],
        out_specs=pl.BlockSpec((1, qchunk, cout), lambda b, q: (b, q, 0)),
        out_shape=jax.ShapeDtypeStruct((_B, s, cout), _F32),
    )(xyzt, gsrc, new_xyz, npad, w1, b1.reshape(1, -1), w2,
      b2.reshape(1, -1), w3, b3.reshape(1, -1))
    return out


def _fp_body(unk_ref, kt_ref, f_ref, w_ref, b_ref, out_ref, *, s):
    unk = unk_ref[0]
    kt = kt_ref[0]
    feats = f_ref[0]
    w = w_ref[:]
    b = b_ref[:]

    t0 = kt[0:1, :]
    t1 = kt[1:2, :]
    t2 = kt[2:3, :]
    nk = t0 * t0 + t1 * t1 + t2 * t2
    nu = jnp.sum(unk * unk, axis=1, keepdims=True)
    d = jnp.dot(unk, kt, preferred_element_type=_F32)
    sqd = (nu + nk) - 2.0 * d
    iota_s = jax.lax.broadcasted_iota(jnp.int32, (_N, s), 1).astype(_F32)

    gs = []
    rs = []
    for _ in range(3):
        m = jnp.min(sqd, axis=1, keepdims=True)
        cand = jnp.where(sqd == m, iota_s, float(s))
        idx = jnp.min(cand, axis=1, keepdims=True)
        oh = iota_s == idx
        ohf = jnp.where(oh, 1.0, 0.0).astype(_F32)
        gs.append(jnp.dot(ohf, feats, preferred_element_type=_F32,
                          precision=jax.lax.Precision.HIGHEST))
        dist = jnp.sqrt(jnp.maximum(m, 1e-12))
        rs.append(1.0 / (dist + 1e-8))
        sqd = jnp.where(oh, 1e30, sqd)

    norm = rs[0] + rs[1] + rs[2]
    interp = (
        gs[0] * (rs[0] / norm) + gs[1] * (rs[1] / norm) + gs[2] * (rs[2] / norm)
    )
    out_ref[0] = _relu(jnp.dot(interp, w, preferred_element_type=_F32) + b)


def _fp(unknown, known, known_feats, layers):
    s = known.shape[1]
    c = known_feats.shape[2]
    (w, b) = layers[0]
    cout = w.shape[1]
    kt = jnp.transpose(known, (0, 2, 1))
    out = pl.pallas_call(
        functools.partial(_fp_body, s=s),
        grid=(_B,),
        in_specs=[
            pl.BlockSpec((1, _N, 3), lambda b_: (b_, 0, 0)),
            pl.BlockSpec((1, 3, s), lambda b_: (b_, 0, 0)),
            pl.BlockSpec((1, s, c), lambda b_: (b_, 0, 0)),
            pl.BlockSpec(w.shape, lambda b_: (0, 0)),
            pl.BlockSpec((1, cout), lambda b_: (0, 0)),
        ],
        out_specs=pl.BlockSpec((1, _N, cout), lambda b_: (b_, 0, 0)),
        out_shape=jax.ShapeDtypeStruct((_B, _N, cout), _F32),
    )(unknown, kt, known_feats, w, b.reshape(1, -1))
    return out


def _fc_body(x_ref, wa1_ref, ba1_ref, wa2_ref, ba2_ref, wb1_ref, bb1_ref,
             wb2_ref, bb2_ref, wp1_ref, bp1_ref, wp2_ref, bp2_ref,
             f0_ref, f1_ref, o0_ref, o1_ref):
    x = x_ref[:]
    f0 = _relu(jnp.dot(x, wa1_ref[:], preferred_element_type=_F32) + ba1_ref[:])
    f0 = _relu(jnp.dot(f0, wa2_ref[:], preferred_element_type=_F32) + ba2_ref[:])
    f1 = _relu(jnp.dot(x, wb1_ref[:], preferred_element_type=_F32) + bb1_ref[:])
    f1 = _relu(jnp.dot(f1, wb2_ref[:], preferred_element_type=_F32) + bb2_ref[:])

    def pcd(f):
        h = _relu(jnp.dot(f, wp1_ref[:], preferred_element_type=_F32) + bp1_ref[:])
        return jnp.dot(h, wp2_ref[:], preferred_element_type=_F32) + bp2_ref[:]

    f0_ref[:] = f0
    f1_ref[:] = f1
    o0_ref[:] = pcd(f0)
    o1_ref[:] = pcd(f1)


def _fc(fused, fc_layers, pcd_layers):
    rows = _B * _N
    chunk = 2048
    cin = fused.shape[-1]
    x = fused.reshape(rows, cin)
    (wa1, ba1), (wa2, ba2) = fc_layers[0]
    (wb1, bb1), (wb2, bb2) = fc_layers[1]
    (wp1, bp1), (wp2, bp2) = pcd_layers
    wspec = lambda a: pl.BlockSpec(a.shape, lambda i: (0, 0))
    bspec = lambda a: pl.BlockSpec((1, a.shape[0]), lambda i: (0, 0))
    outs = pl.pallas_call(
        _fc_body,
        grid=(rows // chunk,),
        in_specs=[
            pl.BlockSpec((chunk, cin), lambda i: (i, 0)),
            wspec(wa1), bspec(ba1), wspec(wa2), bspec(ba2),
            wspec(wb1), bspec(bb1), wspec(wb2), bspec(bb2),
            wspec(wp1), bspec(bp1), wspec(wp2), bspec(bp2),
        ],
        out_specs=[
            pl.BlockSpec((chunk, 4), lambda i: (i, 0)),
            pl.BlockSpec((chunk, 4), lambda i: (i, 0)),
            pl.BlockSpec((chunk, 7), lambda i: (i, 0)),
            pl.BlockSpec((chunk, 7), lambda i: (i, 0)),
        ],
        out_shape=[
            jax.ShapeDtypeStruct((rows, 4), _F32),
            jax.ShapeDtypeStruct((rows, 4), _F32),
            jax.ShapeDtypeStruct((rows, 7), _F32),
            jax.ShapeDtypeStruct((rows, 7), _F32),
        ],
    )(x, wa1, ba1.reshape(1, -1), wa2, ba2.reshape(1, -1),
      wb1, bb1.reshape(1, -1), wb2, bb2.reshape(1, -1),
      wp1, bp1.reshape(1, -1), wp2, bp2.reshape(1, -1))
    f0, f1, o0, o1 = outs
    return f0, f1, o0, o1


_SA_NPOINTS = [1024, 512, 256, 128]
_SA_RADII = [0.05, 0.1, 0.2, 0.3]
_SA_QCHUNK = [256, 256, 256, 128]


def kernel(points, params):
    xyz = points[..., :3]
    feats = points[..., 3:]
    l_xyz = [xyz]
    l_feats = [feats]
    for k in range(4):
        new_xyz = _fps(l_xyz[k], _SA_NPOINTS[k])
        new_feats = _sa(
            l_xyz[k], new_xyz, l_feats[k], params['sa'][k], _SA_RADII[k],
            _SA_QCHUNK[k],
        )
        l_xyz.append(new_xyz)
        l_feats.append(new_feats)
    up = [
        _fp(xyz, l_xyz[k + 2], l_feats[k + 2], params['fp'][k])
        for k in range(3)
    ]
    fused = jnp.concatenate([xyz, l_feats[1]] + up, axis=-1)
    f0, f1, o0, o1 = _fc(fused, params['fc'], params['pcd'])
    r_feats = jnp.concatenate(
        [f0.reshape(_B, _N, 4), f1.reshape(_B, _N, 4)], axis=1
    )
    out = jnp.concatenate(
        [o0.reshape(_B, _N, 7), o1.reshape(_B, _N, 7)], axis=1
    )
    return out, r_feats


# speedup vs baseline: 1.8504x; 1.8504x over previous
import functools

import jax
import jax.numpy as jnp
from jax.experimental import pallas as pl

_B = 16
_N = 1024
_NSAMPLE = 32
_F32 = jnp.float32


def _relu(x):
    return jnp.maximum(x, 0.0)


def _cumsum_lanes(x, width):
    shift = 1
    q = x.shape[0]
    while shift < width:
        z = jnp.zeros((q, shift), dtype=x.dtype)
        x = x + jnp.concatenate([z, x[:, : width - shift]], axis=1)
        shift *= 2
    return x


def _fps_body(xyzt_ref, out_ref, *, n, npoint):
    x0 = xyzt_ref[0]
    x1 = xyzt_ref[1]
    x2 = xyzt_ref[2]
    iota_n = jax.lax.broadcasted_iota(jnp.int32, (_B, n), 1).astype(_F32)
    iota_np = jax.lax.broadcasted_iota(jnp.int32, (_B, npoint), 1).astype(_F32)

    def body(i, carry):
        dists, far, nx0, nx1, nx2 = carry
        onehot = iota_n == far
        c0 = jnp.sum(jnp.where(onehot, x0, 0.0), axis=1, keepdims=True)
        c1 = jnp.sum(jnp.where(onehot, x1, 0.0), axis=1, keepdims=True)
        c2 = jnp.sum(jnp.where(onehot, x2, 0.0), axis=1, keepdims=True)
        fi = i.astype(_F32)
        sel = iota_np == fi
        nx0 = jnp.where(sel, c0, nx0)
        nx1 = jnp.where(sel, c1, nx1)
        nx2 = jnp.where(sel, c2, nx2)
        d0 = x0 - c0
        d1 = x1 - c1
        d2 = x2 - c2
        d = d0 * d0 + d1 * d1 + d2 * d2
        dists = jnp.minimum(dists, d)
        m = jnp.max(dists, axis=1, keepdims=True)
        cand = jnp.where(dists == m, iota_n, float(n))
        far = jnp.min(cand, axis=1, keepdims=True)
        return (dists, far, nx0, nx1, nx2)

    init = (
        jnp.full((_B, n), 1e10, dtype=_F32),
        jnp.zeros((_B, 1), dtype=_F32),
        jnp.zeros((_B, npoint), dtype=_F32),
        jnp.zeros((_B, npoint), dtype=_F32),
        jnp.zeros((_B, npoint), dtype=_F32),
    )
    _, _, nx0, nx1, nx2 = jax.lax.fori_loop(0, npoint, body, init)
    out_ref[0] = nx0
    out_ref[1] = nx1
    out_ref[2] = nx2


def _fps(xyz, npoint):
    n = xyz.shape[1]
    xyzt = jnp.transpose(xyz, (2, 0, 1))
    out = pl.pallas_call(
        functools.partial(_fps_body, n=n, npoint=npoint),
        out_shape=jax.ShapeDtypeStruct((3, _B, npoint), _F32),
    )(xyzt)
    return jnp.transpose(out, (1, 2, 0))


def _sa_body(xyzt_ref, hi_ref, lo_ref, tri_ref, new_ref, npad_ref, w1_ref,
             b1_ref, w2_ref, b2_ref, w3_ref, b3_ref, out_ref, *, r2, n):
    xt = xyzt_ref[0]
    hi = hi_ref[0]
    lo = lo_ref[0]
    tri = tri_ref[:]
    new = new_ref[0]
    npad = npad_ref[0]
    w1 = w1_ref[:]
    b1 = b1_ref[:]
    w2 = w2_ref[:]
    b2 = b2_ref[:]
    w3 = w3_ref[:]
    b3 = b3_ref[:]

    t0 = xt[0:1, :]
    t1 = xt[1:2, :]
    t2 = xt[2:3, :]
    nsrc = t0 * t0 + t1 * t1 + t2 * t2
    nnew = jnp.sum(new * new, axis=1, keepdims=True)
    d = jnp.dot(new, xt, preferred_element_type=_F32)
    sqd = (nnew + nsrc) - 2.0 * d
    mask = sqd <= r2
    mf = jnp.where(mask, 1.0, 0.0).astype(_F32)
    cnt = jnp.dot(mf, tri, preferred_element_type=_F32)
    count = cnt[:, n - 1 : n]

    def gather_slot(kf):
        sel = jnp.where(jnp.logical_and(mask, cnt == kf), 1.0, 0.0)
        sel = sel.astype(_F32)
        return (jnp.dot(sel, hi, preferred_element_type=_F32)
                + jnp.dot(sel, lo, preferred_element_type=_F32))

    def mlp3(h):
        a = _relu(jnp.dot(h, w1, preferred_element_type=_F32) + b1)
        a = _relu(jnp.dot(a, w2, preferred_element_type=_F32) + b2)
        a = _relu(jnp.dot(a, w3, preferred_element_type=_F32) + b3)
        return a

    g0 = gather_slot(1.0)
    g0 = jnp.where(count >= 1.0, g0, hi[0:1, :] + lo[0:1, :])
    a0 = mlp3(g0 - npad)

    def body(k, carry):
        acc, g0 = carry
        kf = k.astype(_F32) + 1.0
        g = gather_slot(kf)
        g = jnp.where(count >= kf, g, g0)
        a = mlp3(g - npad)
        return (jnp.maximum(acc, a), g0)

    acc, _ = jax.lax.fori_loop(1, _NSAMPLE, body, (a0, g0))
    out_ref[0] = acc


def _sa(src_xyz, new_xyz, feats, layers, radius, qchunk):
    n = src_xyz.shape[1]
    s = new_xyz.shape[1]
    c = feats.shape[2]
    ch = 3 + c
    (w1, b1), (w2, b2), (w3, b3) = layers
    cout = w3.shape[1]
    gsrc = jnp.concatenate([src_xyz, feats], axis=-1)
    hi = gsrc.astype(jnp.bfloat16).astype(_F32)
    lo = gsrc - hi
    tri = jnp.where(
        jax.lax.broadcasted_iota(jnp.int32, (n, n), 0)
        <= jax.lax.broadcasted_iota(jnp.int32, (n, n), 1),
        1.0, 0.0).astype(_F32)
    xyzt = jnp.transpose(src_xyz, (0, 2, 1))
    npad = jnp.concatenate(
        [new_xyz, jnp.zeros((_B, s, c), dtype=_F32)], axis=-1
    )
    nq = s // qchunk
    grid = (_B, nq)
    out = pl.pallas_call(
        functools.partial(_sa_body, r2=radius * radius, n=n),
        grid=grid,
        in_specs=[
            pl.BlockSpec((1, 3, n), lambda b, q: (b, 0, 0)),
            pl.BlockSpec((1, n, ch), lambda b, q: (b, 0, 0)),
            pl.BlockSpec((1, n, ch), lambda b, q: (b, 0, 0)),
            pl.BlockSpec((n, n), lambda b, q: (0, 0)),
            pl.BlockSpec((1, qchunk, 3), lambda b, q: (b, q, 0)),
            pl.BlockSpec((1, qchunk, ch), lambda b, q: (b, q, 0)),
            pl.BlockSpec(w1.shape, lambda b, q: (0, 0)),
            pl.BlockSpec((1, b1.shape[0]), lambda b, q: (0, 0)),
            pl.BlockSpec(w2.shape, lambda b, q: (0, 0)),
            pl.BlockSpec((1, b2.shape[0]), lambda b, q: (0, 0)),
            pl.BlockSpec(w3.shape, lambda b, q: (0, 0)),
            pl.BlockSpec((1, b3.shape[0]), lambda b, q: (0, 0)),
        ],
        out_specs=pl.BlockSpec((1, qchunk, cout), lambda b, q: (b, q, 0)),
        out_shape=jax.ShapeDtypeStruct((_B, s, cout), _F32),
    )(xyzt, hi, lo, tri, new_xyz, npad, w1, b1.reshape(1, -1), w2,
      b2.reshape(1, -1), w3, b3.reshape(1, -1))
    return out


def _fp_body(unk_ref, kt_ref, f_ref, w_ref, b_ref, out_ref, *, s):
    unk = unk_ref[0]
    kt = kt_ref[0]
    feats = f_ref[0]
    w = w_ref[:]
    b = b_ref[:]

    t0 = kt[0:1, :]
    t1 = kt[1:2, :]
    t2 = kt[2:3, :]
    nk = t0 * t0 + t1 * t1 + t2 * t2
    nu = jnp.sum(unk * unk, axis=1, keepdims=True)
    d = jnp.dot(unk, kt, preferred_element_type=_F32)
    sqd = (nu + nk) - 2.0 * d
    iota_s = jax.lax.broadcasted_iota(jnp.int32, (_N, s), 1).astype(_F32)

    gs = []
    rs = []
    for _ in range(3):
        m = jnp.min(sqd, axis=1, keepdims=True)
        cand = jnp.where(sqd == m, iota_s, float(s))
        idx = jnp.min(cand, axis=1, keepdims=True)
        oh = iota_s == idx
        ohf = jnp.where(oh, 1.0, 0.0).astype(_F32)
        gs.append(jnp.dot(ohf, feats, preferred_element_type=_F32,
                          precision=jax.lax.Precision.HIGHEST))
        dist = jnp.sqrt(jnp.maximum(m, 1e-12))
        rs.append(1.0 / (dist + 1e-8))
        sqd = jnp.where(oh, 1e30, sqd)

    norm = rs[0] + rs[1] + rs[2]
    interp = (
        gs[0] * (rs[0] / norm) + gs[1] * (rs[1] / norm) + gs[2] * (rs[2] / norm)
    )
    out_ref[0] = _relu(jnp.dot(interp, w, preferred_element_type=_F32) + b)


def _fp(unknown, known, known_feats, layers):
    s = known.shape[1]
    c = known_feats.shape[2]
    (w, b) = layers[0]
    cout = w.shape[1]
    kt = jnp.transpose(known, (0, 2, 1))
    out = pl.pallas_call(
        functools.partial(_fp_body, s=s),
        grid=(_B,),
        in_specs=[
            pl.BlockSpec((1, _N, 3), lambda b_: (b_, 0, 0)),
            pl.BlockSpec((1, 3, s), lambda b_: (b_, 0, 0)),
            pl.BlockSpec((1, s, c), lambda b_: (b_, 0, 0)),
            pl.BlockSpec(w.shape, lambda b_: (0, 0)),
            pl.BlockSpec((1, cout), lambda b_: (0, 0)),
        ],
        out_specs=pl.BlockSpec((1, _N, cout), lambda b_: (b_, 0, 0)),
        out_shape=jax.ShapeDtypeStruct((_B, _N, cout), _F32),
    )(unknown, kt, known_feats, w, b.reshape(1, -1))
    return out


def _fc_body(x_ref, wa1_ref, ba1_ref, wa2_ref, ba2_ref, wb1_ref, bb1_ref,
             wb2_ref, bb2_ref, wp1_ref, bp1_ref, wp2_ref, bp2_ref,
             f0_ref, f1_ref, o0_ref, o1_ref):
    x = x_ref[:]
    f0 = _relu(jnp.dot(x, wa1_ref[:], preferred_element_type=_F32) + ba1_ref[:])
    f0 = _relu(jnp.dot(f0, wa2_ref[:], preferred_element_type=_F32) + ba2_ref[:])
    f1 = _relu(jnp.dot(x, wb1_ref[:], preferred_element_type=_F32) + bb1_ref[:])
    f1 = _relu(jnp.dot(f1, wb2_ref[:], preferred_element_type=_F32) + bb2_ref[:])

    def pcd(f):
        h = _relu(jnp.dot(f, wp1_ref[:], preferred_element_type=_F32) + bp1_ref[:])
        return jnp.dot(h, wp2_ref[:], preferred_element_type=_F32) + bp2_ref[:]

    f0_ref[:] = f0
    f1_ref[:] = f1
    o0_ref[:] = pcd(f0)
    o1_ref[:] = pcd(f1)


def _fc(fused, fc_layers, pcd_layers):
    rows = _B * _N
    chunk = 2048
    cin = fused.shape[-1]
    x = fused.reshape(rows, cin)
    (wa1, ba1), (wa2, ba2) = fc_layers[0]
    (wb1, bb1), (wb2, bb2) = fc_layers[1]
    (wp1, bp1), (wp2, bp2) = pcd_layers
    wspec = lambda a: pl.BlockSpec(a.shape, lambda i: (0, 0))
    bspec = lambda a: pl.BlockSpec((1, a.shape[0]), lambda i: (0, 0))
    outs = pl.pallas_call(
        _fc_body,
        grid=(rows // chunk,),
        in_specs=[
            pl.BlockSpec((chunk, cin), lambda i: (i, 0)),
            wspec(wa1), bspec(ba1), wspec(wa2), bspec(ba2),
            wspec(wb1), bspec(bb1), wspec(wb2), bspec(bb2),
            wspec(wp1), bspec(bp1), wspec(wp2), bspec(bp2),
        ],
        out_specs=[
            pl.BlockSpec((chunk, 4), lambda i: (i, 0)),
            pl.BlockSpec((chunk, 4), lambda i: (i, 0)),
            pl.BlockSpec((chunk, 7), lambda i: (i, 0)),
            pl.BlockSpec((chunk, 7), lambda i: (i, 0)),
        ],
        out_shape=[
            jax.ShapeDtypeStruct((rows, 4), _F32),
            jax.ShapeDtypeStruct((rows, 4), _F32),
            jax.ShapeDtypeStruct((rows, 7), _F32),
            jax.ShapeDtypeStruct((rows, 7), _F32),
        ],
    )(x, wa1, ba1.reshape(1, -1), wa2, ba2.reshape(1, -1),
      wb1, bb1.reshape(1, -1), wb2, bb2.reshape(1, -1),
      wp1, bp1.reshape(1, -1), wp2, bp2.reshape(1, -1))
    f0, f1, o0, o1 = outs
    return f0, f1, o0, o1


_SA_NPOINTS = [1024, 512, 256, 128]
_SA_RADII = [0.05, 0.1, 0.2, 0.3]
_SA_QCHUNK = [256, 256, 256, 128]


def kernel(points, params):
    xyz = points[..., :3]
    feats = points[..., 3:]
    l_xyz = [xyz]
    l_feats = [feats]
    for k in range(4):
        new_xyz = _fps(l_xyz[k], _SA_NPOINTS[k])
        new_feats = _sa(
            l_xyz[k], new_xyz, l_feats[k], params['sa'][k], _SA_RADII[k],
            _SA_QCHUNK[k],
        )
        l_xyz.append(new_xyz)
        l_feats.append(new_feats)
    up = [
        _fp(xyz, l_xyz[k + 2], l_feats[k + 2], params['fp'][k])
        for k in range(3)
    ]
    fused = jnp.concatenate([xyz, l_feats[1]] + up, axis=-1)
    f0, f1, o0, o1 = _fc(fused, params['fc'], params['pcd'])
    r_feats = jnp.concatenate(
        [f0.reshape(_B, _N, 4), f1.reshape(_B, _N, 4)], axis=1
    )
    out = jnp.concatenate(
        [o0.reshape(_B, _N, 7), o1.reshape(_B, _N, 7)], axis=1
    )
    return out, r_feats


# speedup vs baseline: 1.9175x; 1.0363x over previous
import functools

import jax
import jax.numpy as jnp
from jax.experimental import pallas as pl

_B = 16
_N = 1024
_NSAMPLE = 32
_F32 = jnp.float32


def _relu(x):
    return jnp.maximum(x, 0.0)


def _cumsum_lanes(x, width):
    shift = 1
    q = x.shape[0]
    while shift < width:
        z = jnp.zeros((q, shift), dtype=x.dtype)
        x = x + jnp.concatenate([z, x[:, : width - shift]], axis=1)
        shift *= 2
    return x


def _fps_body(xyzt_ref, out_ref, *, n, npoint):
    x0 = xyzt_ref[0]
    x1 = xyzt_ref[1]
    x2 = xyzt_ref[2]
    iota_n = jax.lax.broadcasted_iota(jnp.int32, (_B, n), 1).astype(_F32)
    iota_np = jax.lax.broadcasted_iota(jnp.int32, (_B, npoint), 1).astype(_F32)

    def body(i, carry):
        dists, far, nx0, nx1, nx2 = carry
        onehot = iota_n == far
        c0 = jnp.sum(jnp.where(onehot, x0, 0.0), axis=1, keepdims=True)
        c1 = jnp.sum(jnp.where(onehot, x1, 0.0), axis=1, keepdims=True)
        c2 = jnp.sum(jnp.where(onehot, x2, 0.0), axis=1, keepdims=True)
        fi = i.astype(_F32)
        sel = iota_np == fi
        nx0 = jnp.where(sel, c0, nx0)
        nx1 = jnp.where(sel, c1, nx1)
        nx2 = jnp.where(sel, c2, nx2)
        d0 = x0 - c0
        d1 = x1 - c1
        d2 = x2 - c2
        d = d0 * d0 + d1 * d1 + d2 * d2
        dists = jnp.minimum(dists, d)
        m = jnp.max(dists, axis=1, keepdims=True)
        cand = jnp.where(dists == m, iota_n, float(n))
        far = jnp.min(cand, axis=1, keepdims=True)
        return (dists, far, nx0, nx1, nx2)

    init = (
        jnp.full((_B, n), 1e10, dtype=_F32),
        jnp.zeros((_B, 1), dtype=_F32),
        jnp.zeros((_B, npoint), dtype=_F32),
        jnp.zeros((_B, npoint), dtype=_F32),
        jnp.zeros((_B, npoint), dtype=_F32),
    )
    _, _, nx0, nx1, nx2 = jax.lax.fori_loop(0, npoint, body, init)
    out_ref[0] = nx0
    out_ref[1] = nx1
    out_ref[2] = nx2


def _fps(xyz, npoint):
    n = xyz.shape[1]
    xyzt = jnp.transpose(xyz, (2, 0, 1))
    out = pl.pallas_call(
        functools.partial(_fps_body, n=n, npoint=npoint),
        out_shape=jax.ShapeDtypeStruct((3, _B, npoint), _F32),
    )(xyzt)
    return jnp.transpose(out, (1, 2, 0))


def _sa_body(xyzt_ref, hsrc_ref, tri_ref, new_ref, npad_ref, w1_ref,
             b1_ref, w2_ref, b2_ref, w3_ref, b3_ref, out_ref, *, r2, n, ch):
    xt = xyzt_ref[0]
    hsrc = hsrc_ref[0]
    tri = tri_ref[:]
    new = new_ref[0]
    npad = npad_ref[0]
    w1 = w1_ref[:]
    b1 = b1_ref[:]
    w2 = w2_ref[:]
    b2 = b2_ref[:]
    w3 = w3_ref[:]
    b3 = b3_ref[:]

    t0 = xt[0:1, :]
    t1 = xt[1:2, :]
    t2 = xt[2:3, :]
    nsrc = t0 * t0 + t1 * t1 + t2 * t2
    nnew = jnp.sum(new * new, axis=1, keepdims=True)
    d = jnp.dot(new, xt, preferred_element_type=_F32)
    sqd = (nnew + nsrc) - 2.0 * d
    mask = sqd <= r2
    mf = jnp.where(mask, 1.0, 0.0).astype(_F32)
    cnt = jnp.dot(mf, tri, preferred_element_type=_F32)
    count = cnt[:, n - 1 : n]

    def fold(m):
        return (m[:, :ch] + m[:, ch : 2 * ch]) + m[:, 2 * ch :]

    def gather_slot(kf):
        sel = jnp.where(jnp.logical_and(mask, cnt == kf), 1.0, 0.0)
        sel = sel.astype(_F32)
        return fold(jnp.dot(sel, hsrc, preferred_element_type=_F32))

    def mlp3(h):
        a = _relu(jnp.dot(h, w1, preferred_element_type=_F32) + b1)
        a = _relu(jnp.dot(a, w2, preferred_element_type=_F32) + b2)
        a = _relu(jnp.dot(a, w3, preferred_element_type=_F32) + b3)
        return a

    g0 = gather_slot(1.0)
    g0 = jnp.where(count >= 1.0, g0, fold(hsrc[0:1, :]))
    a0 = mlp3(g0 - npad)

    def body(k, carry):
        acc, g0 = carry
        kf = k.astype(_F32) + 1.0
        g = gather_slot(kf)
        g = jnp.where(count >= kf, g, g0)
        a = mlp3(g - npad)
        return (jnp.maximum(acc, a), g0)

    acc, _ = jax.lax.fori_loop(1, _NSAMPLE, body, (a0, g0))
    out_ref[0] = acc


def _sa(src_xyz, new_xyz, feats, layers, radius, qchunk):
    n = src_xyz.shape[1]
    s = new_xyz.shape[1]
    c = feats.shape[2]
    ch = 3 + c
    (w1, b1), (w2, b2), (w3, b3) = layers
    cout = w3.shape[1]
    gsrc = jnp.concatenate([src_xyz, feats], axis=-1)
    h1 = gsrc.astype(jnp.bfloat16).astype(_F32)
    r = gsrc - h1
    h2 = r.astype(jnp.bfloat16).astype(_F32)
    h3 = r - h2
    hsrc = jnp.concatenate([h1, h2, h3], axis=-1)
    tri = jnp.where(
        jax.lax.broadcasted_iota(jnp.int32, (n, n), 0)
        <= jax.lax.broadcasted_iota(jnp.int32, (n, n), 1),
        1.0, 0.0).astype(_F32)
    xyzt = jnp.transpose(src_xyz, (0, 2, 1))
    npad = jnp.concatenate(
        [new_xyz, jnp.zeros((_B, s, c), dtype=_F32)], axis=-1
    )
    nq = s // qchunk
    grid = (_B, nq)
    out = pl.pallas_call(
        functools.partial(_sa_body, r2=radius * radius, n=n, ch=ch),
        grid=grid,
        in_specs=[
            pl.BlockSpec((1, 3, n), lambda b, q: (b, 0, 0)),
            pl.BlockSpec((1, n, 3 * ch), lambda b, q: (b, 0, 0)),
            pl.BlockSpec((n, n), lambda b, q: (0, 0)),
            pl.BlockSpec((1, qchunk, 3), lambda b, q: (b, q, 0)),
            pl.BlockSpec((1, qchunk, ch), lambda b, q: (b, q, 0)),
            pl.BlockSpec(w1.shape, lambda b, q: (0, 0)),
            pl.BlockSpec((1, b1.shape[0]), lambda b, q: (0, 0)),
            pl.BlockSpec(w2.shape, lambda b, q: (0, 0)),
            pl.BlockSpec((1, b2.shape[0]), lambda b, q: (0, 0)),
            pl.BlockSpec(w3.shape, lambda b, q: (0, 0)),
            pl.BlockSpec((1, b3.shape[0]), lambda b, q: (0, 0)),
        ],
        out_specs=pl.BlockSpec((1, qchunk, cout), lambda b, q: (b, q, 0)),
        out_shape=jax.ShapeDtypeStruct((_B, s, cout), _F32),
    )(xyzt, hsrc, tri, new_xyz, npad, w1, b1.reshape(1, -1), w2,
      b2.reshape(1, -1), w3, b3.reshape(1, -1))
    return out


def _fp_body(unk_ref, kt_ref, f_ref, w_ref, b_ref, out_ref, *, s):
    unk = unk_ref[0]
    kt = kt_ref[0]
    feats = f_ref[0]
    w = w_ref[:]
    b = b_ref[:]

    t0 = kt[0:1, :]
    t1 = kt[1:2, :]
    t2 = kt[2:3, :]
    nk = t0 * t0 + t1 * t1 + t2 * t2
    nu = jnp.sum(unk * unk, axis=1, keepdims=True)
    d = jnp.dot(unk, kt, preferred_element_type=_F32)
    sqd = (nu + nk) - 2.0 * d
    iota_s = jax.lax.broadcasted_iota(jnp.int32, (_N, s), 1).astype(_F32)

    gs = []
    rs = []
    for _ in range(3):
        m = jnp.min(sqd, axis=1, keepdims=True)
        cand = jnp.where(sqd == m, iota_s, float(s))
        idx = jnp.min(cand, axis=1, keepdims=True)
        oh = iota_s == idx
        ohf = jnp.where(oh, 1.0, 0.0).astype(_F32)
        gs.append(jnp.dot(ohf, feats, preferred_element_type=_F32,
                          precision=jax.lax.Precision.HIGHEST))
        dist = jnp.sqrt(jnp.maximum(m, 1e-12))
        rs.append(1.0 / (dist + 1e-8))
        sqd = jnp.where(oh, 1e30, sqd)

    norm = rs[0] + rs[1] + rs[2]
    interp = (
        gs[0] * (rs[0] / norm) + gs[1] * (rs[1] / norm) + gs[2] * (rs[2] / norm)
    )
    out_ref[0] = _relu(jnp.dot(interp, w, preferred_element_type=_F32) + b)


def _fp(unknown, known, known_feats, layers):
    s = known.shape[1]
    c = known_feats.shape[2]
    (w, b) = layers[0]
    cout = w.shape[1]
    kt = jnp.transpose(known, (0, 2, 1))
    out = pl.pallas_call(
        functools.partial(_fp_body, s=s),
        grid=(_B,),
        in_specs=[
            pl.BlockSpec((1, _N, 3), lambda b_: (b_, 0, 0)),
            pl.BlockSpec((1, 3, s), lambda b_: (b_, 0, 0)),
            pl.BlockSpec((1, s, c), lambda b_: (b_, 0, 0)),
            pl.BlockSpec(w.shape, lambda b_: (0, 0)),
            pl.BlockSpec((1, cout), lambda b_: (0, 0)),
        ],
        out_specs=pl.BlockSpec((1, _N, cout), lambda b_: (b_, 0, 0)),
        out_shape=jax.ShapeDtypeStruct((_B, _N, cout), _F32),
    )(unknown, kt, known_feats, w, b.reshape(1, -1))
    return out


def _fc_body(x_ref, wa1_ref, ba1_ref, wa2_ref, ba2_ref, wb1_ref, bb1_ref,
             wb2_ref, bb2_ref, wp1_ref, bp1_ref, wp2_ref, bp2_ref,
             f0_ref, f1_ref, o0_ref, o1_ref):
    x = x_ref[:]
    f0 = _relu(jnp.dot(x, wa1_ref[:], preferred_element_type=_F32) + ba1_ref[:])
    f0 = _relu(jnp.dot(f0, wa2_ref[:], preferred_element_type=_F32) + ba2_ref[:])
    f1 = _relu(jnp.dot(x, wb1_ref[:], preferred_element_type=_F32) + bb1_ref[:])
    f1 = _relu(jnp.dot(f1, wb2_ref[:], preferred_element_type=_F32) + bb2_ref[:])

    def pcd(f):
        h = _relu(jnp.dot(f, wp1_ref[:], preferred_element_type=_F32) + bp1_ref[:])
        return jnp.dot(h, wp2_ref[:], preferred_element_type=_F32) + bp2_ref[:]

    f0_ref[:] = f0
    f1_ref[:] = f1
    o0_ref[:] = pcd(f0)
    o1_ref[:] = pcd(f1)


def _fc(fused, fc_layers, pcd_layers):
    rows = _B * _N
    chunk = 2048
    cin = fused.shape[-1]
    x = fused.reshape(rows, cin)
    (wa1, ba1), (wa2, ba2) = fc_layers[0]
    (wb1, bb1), (wb2, bb2) = fc_layers[1]
    (wp1, bp1), (wp2, bp2) = pcd_layers
    wspec = lambda a: pl.BlockSpec(a.shape, lambda i: (0, 0))
    bspec = lambda a: pl.BlockSpec((1, a.shape[0]), lambda i: (0, 0))
    outs = pl.pallas_call(
        _fc_body,
        grid=(rows // chunk,),
        in_specs=[
            pl.BlockSpec((chunk, cin), lambda i: (i, 0)),
            wspec(wa1), bspec(ba1), wspec(wa2), bspec(ba2),
            wspec(wb1), bspec(bb1), wspec(wb2), bspec(bb2),
            wspec(wp1), bspec(bp1), wspec(wp2), bspec(bp2),
        ],
        out_specs=[
            pl.BlockSpec((chunk, 4), lambda i: (i, 0)),
            pl.BlockSpec((chunk, 4), lambda i: (i, 0)),
            pl.BlockSpec((chunk, 7), lambda i: (i, 0)),
            pl.BlockSpec((chunk, 7), lambda i: (i, 0)),
        ],
        out_shape=[
            jax.ShapeDtypeStruct((rows, 4), _F32),
            jax.ShapeDtypeStruct((rows, 4), _F32),
            jax.ShapeDtypeStruct((rows, 7), _F32),
            jax.ShapeDtypeStruct((rows, 7), _F32),
        ],
    )(x, wa1, ba1.reshape(1, -1), wa2, ba2.reshape(1, -1),
      wb1, bb1.reshape(1, -1), wb2, bb2.reshape(1, -1),
      wp1, bp1.reshape(1, -1), wp2, bp2.reshape(1, -1))
    f0, f1, o0, o1 = outs
    return f0, f1, o0, o1


_SA_NPOINTS = [1024, 512, 256, 128]
_SA_RADII = [0.05, 0.1, 0.2, 0.3]
_SA_QCHUNK = [256, 256, 256, 128]


def kernel(points, params):
    xyz = points[..., :3]
    feats = points[..., 3:]
    l_xyz = [xyz]
    l_feats = [feats]
    for k in range(4):
        new_xyz = _fps(l_xyz[k], _SA_NPOINTS[k])
        new_feats = _sa(
            l_xyz[k], new_xyz, l_feats[k], params['sa'][k], _SA_RADII[k],
            _SA_QCHUNK[k],
        )
        l_xyz.append(new_xyz)
        l_feats.append(new_feats)
    up = [
        _fp(xyz, l_xyz[k + 2], l_feats[k + 2], params['fp'][k])
        for k in range(3)
    ]
    fused = jnp.concatenate([xyz, l_feats[1]] + up, axis=-1)
    f0, f1, o0, o1 = _fc(fused, params['fc'], params['pcd'])
    r_feats = jnp.concatenate(
        [f0.reshape(_B, _N, 4), f1.reshape(_B, _N, 4)], axis=1
    )
    out = jnp.concatenate(
        [o0.reshape(_B, _N, 7), o1.reshape(_B, _N, 7)], axis=1
    )
    return out, r_feats
